# Initial kernel scaffold; baseline (speedup 1.0000x reference)
import jax
import jax.numpy as jnp
from jax.experimental import pallas as pl


def kernel(x, edge_index, fem_reward, topology_reward, gW1, gb1, gW2, gb2, fW0, fb0, fW1, fb1, fWo, fbo, tW0, tb0, tW1, tb1, tWo, tbo, uW, ub, rW1, rb1, rW2, rb2, aW1, ab1, aW2, ab2):
    raise NotImplementedError("write your pallas kernel here")


# speedup vs baseline: 12.2395x; 12.2395x over previous
import functools

import jax
import jax.numpy as jnp
from jax import lax
from jax.experimental import pallas as pl
from jax.experimental.pallas import tpu as pltpu
from jax.experimental.pallas import tpu_sc as plsc

N = 10000
E = 320000
D_IN = 128
H = 128
OUT = 64
FUS = 128

NC = 2
NS = 16
EPW = E // (NC * NS)
CHUNK = 80
NCHUNK = EPW // CHUNK
NPAD = 10240
RPT = NPAD // NS
DEGW = 16

BLK = 1000
GRID = N // BLK


@functools.cache
def _deg_kernel_build():
    mesh = plsc.VectorSubcoreMesh(core_axis_name="c", subcore_axis_name="s")
    return functools.partial(
        pl.kernel,
        out_type=jax.ShapeDtypeStruct((NC * NPAD,), jnp.float32),
        mesh=mesh,
        scratch_types=[
            pltpu.VMEM((CHUNK,), jnp.int32),
            pltpu.VMEM((CHUNK,), jnp.float32),
            pltpu.VMEM_SHARED((NPAD,), jnp.float32),
        ],
    )(_deg_body)


def _deg_body(dst_hbm, ones_hbm, zeros_hbm, out_hbm, idx_v, ones_v, acc_sh):
    c = lax.axis_index("c")
    s = lax.axis_index("s")
    row0 = s * RPT
    pltpu.sync_copy(zeros_hbm, acc_sh.at[pl.ds(row0, RPT)])
    pltpu.sync_copy(ones_hbm, ones_v)
    plsc.subcore_barrier()
    base = (c * NS + s) * EPW

    def body(k, carry):
        off = base + k * CHUNK
        pltpu.sync_copy(dst_hbm.at[pl.ds(off, CHUNK)], idx_v)
        pltpu.sync_copy(ones_v, acc_sh.at[idx_v], add=True)
        return carry

    lax.fori_loop(0, NCHUNK, body, 0)
    plsc.subcore_barrier()
    pltpu.sync_copy(acc_sh.at[pl.ds(row0, RPT)],
                    out_hbm.at[pl.ds(c * NPAD + row0, RPT)])


@functools.cache
def _agg_kernel_build():
    mesh = plsc.VectorSubcoreMesh(core_axis_name="c", subcore_axis_name="s")
    return functools.partial(
        pl.kernel,
        out_type=jax.ShapeDtypeStruct((NC * NPAD, H), jnp.float32),
        mesh=mesh,
        scratch_types=[
            pltpu.VMEM((CHUNK,), jnp.int32),
            pltpu.VMEM((CHUNK,), jnp.int32),
            pltpu.VMEM((CHUNK, H), jnp.float32),
            pltpu.VMEM_SHARED((NPAD, H), jnp.float32),
            pltpu.SemaphoreType.DMA,
        ],
    )(_agg_body)


def _agg_body(src_hbm, dst_hbm, hs_hbm, zeros_hbm, out_hbm,
              src_v, dst_v, rows_v, acc_sh, sem):
    c = lax.axis_index("c")
    s = lax.axis_index("s")
    row0 = s * RPT
    pltpu.sync_copy(zeros_hbm, acc_sh.at[pl.ds(row0, RPT)])
    plsc.subcore_barrier()
    base = (c * NS + s) * EPW

    def body(k, carry):
        off = base + k * CHUNK
        pltpu.sync_copy(src_hbm.at[pl.ds(off, CHUNK)], src_v)
        pltpu.sync_copy(dst_hbm.at[pl.ds(off, CHUNK)], dst_v)
        pltpu.async_copy(hs_hbm.at[src_v], rows_v, sem).wait()
        pltpu.sync_copy(rows_v, acc_sh.at[dst_v], add=True)
        return carry

    lax.fori_loop(0, NCHUNK, body, 0)
    plsc.subcore_barrier()
    pltpu.sync_copy(acc_sh.at[pl.ds(row0, RPT)],
                    out_hbm.at[pl.ds(c * NPAD + row0, RPT)])


def _dinv(dega, degb):
    return lax.rsqrt(dega[:, :1] + degb[:, :1] + 1.0)


def _tc1_body(x_ref, dega_ref, degb_ref, w_ref, out_ref):
    dinv = _dinv(dega_ref[...], degb_ref[...])
    out_ref[...] = jnp.dot(x_ref[...], w_ref[...],
                           preferred_element_type=jnp.float32) * dinv


def _tc2_body(a1a_ref, a1b_ref, hs1_ref, dega_ref, degb_ref, gb1_ref, gw2_ref,
              out_ref):
    dinv = _dinv(dega_ref[...], degb_ref[...])
    h1 = jnp.maximum((a1a_ref[...] + a1b_ref[...] + hs1_ref[...]) * dinv
                     + gb1_ref[...], 0.0)
    out_ref[...] = jnp.dot(h1, gw2_ref[...],
                           preferred_element_type=jnp.float32) * dinv


def _tc3_body(a2a_ref, a2b_ref, hs2_ref, dega_ref, degb_ref, fem_ref, topo_ref,
              gb2_ref, fw0a_ref, fw0b_ref, fb0_ref, fw1_ref, fb1_ref,
              fwo_ref, fbo_ref, tw0a_ref, tw0b_ref, tb0_ref, tw1_ref, tb1_ref,
              two_ref, tbo_ref, uwh_ref, uwf_ref, uwt_ref, ub_ref,
              rw1_ref, rb1_ref, rw2_ref, rb2_ref, aw1_ref, ab1_ref,
              aw2_ref, ab2_ref, add_ref, rem_ref):
    mm = functools.partial(jnp.dot, preferred_element_type=jnp.float32)
    dinv = _dinv(dega_ref[...], degb_ref[...])
    h = jnp.maximum((a2a_ref[...] + a2b_ref[...] + hs2_ref[...]) * dinv
                    + gb2_ref[...], 0.0)
    f = jnp.maximum(mm(h, fw0a_ref[...]) + fem_ref[...] * fw0b_ref[...]
                    + fb0_ref[...], 0.0)
    f = jnp.maximum(mm(f, fw1_ref[...]) + fb1_ref[...], 0.0)
    f = mm(f, fwo_ref[...]) + fbo_ref[...]
    t = jnp.maximum(mm(h, tw0a_ref[...]) + topo_ref[...] * tw0b_ref[...]
                    + tb0_ref[...], 0.0)
    t = jnp.maximum(mm(t, tw1_ref[...]) + tb1_ref[...], 0.0)
    t = mm(t, two_ref[...]) + tbo_ref[...]
    fus = jnp.maximum(mm(h, uwh_ref[...]) + mm(f, uwf_ref[...])
                      + mm(t, uwt_ref[...]) + ub_ref[...], 0.0)
    r = jnp.maximum(mm(fus, rw1_ref[...]) + rb1_ref[...], 0.0)
    rem_ref[...] = jnp.tanh(mm(r, rw2_ref[...]) + rb2_ref[...])
    a = jnp.maximum(mm(fus, aw1_ref[...]) + ab1_ref[...], 0.0)
    add_ref[...] = jnp.tanh(mm(a, aw2_ref[...]) + ab2_ref[...])


def _row_spec(width):
    return pl.BlockSpec((BLK, width), lambda i: (i, 0))


def _full_spec(shape):
    return pl.BlockSpec(shape, lambda i: (0,) * len(shape))


def _tc_call(body, ins, row_widths, full_shapes, out_widths):
    in_specs = ([_row_spec(w) for w in row_widths]
                + [_full_spec(s) for s in full_shapes])
    out_specs = [_row_spec(w) for w in out_widths]
    out_shape = [jax.ShapeDtypeStruct((N, w), jnp.float32) for w in out_widths]
    outs = pl.pallas_call(
        body,
        grid=(GRID,),
        in_specs=in_specs,
        out_specs=out_specs if len(out_specs) > 1 else out_specs[0],
        out_shape=out_shape if len(out_shape) > 1 else out_shape[0],
    )(*ins)
    return outs


def kernel(x, edge_index, fem_reward, topology_reward, gW1, gb1, gW2, gb2,
           fW0, fb0, fW1, fb1, fWo, fbo, tW0, tb0, tW1, tb1, tWo, tbo,
           uW, ub, rW1, rb1, rW2, rb2, aW1, ab1, aW2, ab2):
    src = edge_index[0]
    dst = edge_index[1]
    zeros_h = jnp.zeros((RPT, H), jnp.float32)
    zeros_d = jnp.zeros((RPT,), jnp.float32)
    ones_d = jnp.ones((CHUNK,), jnp.float32)

    deg_parts = _deg_kernel_build()(dst, ones_d, zeros_d)
    dega = deg_parts[:N].reshape(N, 1)
    degb = deg_parts[NPAD:NPAD + N].reshape(N, 1)

    h1s = _tc_call(_tc1_body, (x, dega, degb, gW1), (D_IN, 1, 1),
                   ((D_IN, H),), (H,))
    agg1 = _agg_kernel_build()(src, dst, h1s, zeros_h)
    a1a = agg1[:N]
    a1b = agg1[NPAD:NPAD + N]

    h2s = _tc_call(_tc2_body,
                   (a1a, a1b, h1s, dega, degb, gb1.reshape(1, H), gW2),
                   (H, H, H, 1, 1), ((1, H), (H, H)), (H,))
    agg2 = _agg_kernel_build()(src, dst, h2s, zeros_h)
    a2a = agg2[:N]
    a2b = agg2[NPAD:NPAD + N]

    fW0a, fW0b = fW0[:H], fW0[H:H + 1]
    tW0a, tW0b = tW0[:H], tW0[H:H + 1]
    uWh, uWf, uWt = uW[:H], uW[H:H + OUT], uW[H + OUT:]
    rW2p = jnp.pad(rW2, ((0, 0), (0, 7)))
    rb2p = jnp.pad(rb2, (0, 7)).reshape(1, 8)
    aW2p = jnp.pad(aW2, ((0, 0), (0, 5)))
    ab2p = jnp.pad(ab2, (0, 5)).reshape(1, 8)

    add_p, rem_p = _tc_call(
        _tc3_body,
        (a2a, a2b, h2s, dega, degb, fem_reward, topology_reward,
         gb2.reshape(1, H),
         fW0a, fW0b, fb0.reshape(1, H), fW1, fb1.reshape(1, H),
         fWo, fbo.reshape(1, OUT),
         tW0a, tW0b, tb0.reshape(1, H), tW1, tb1.reshape(1, H),
         tWo, tbo.reshape(1, OUT),
         uWh, uWf, uWt, ub.reshape(1, FUS),
         rW1, rb1.reshape(1, FUS), rW2p, rb2p,
         aW1, ab1.reshape(1, FUS), aW2p, ab2p),
        (H, H, H, 1, 1, 1, 1),
        ((1, H), (H, H), (1, H), (1, H), (H, H), (1, H), (H, OUT), (1, OUT),
         (H, H), (1, H), (1, H), (H, H), (1, H), (H, OUT), (1, OUT),
         (H, FUS), (OUT, FUS), (OUT, FUS), (1, FUS),
         (FUS, FUS), (1, FUS), (FUS, 8), (1, 8),
         (FUS, FUS), (1, FUS), (FUS, 8), (1, 8)),
        (8, 8))
    return (add_p[:, :3], rem_p[:, :1])


# speedup vs baseline: 25.2996x; 2.0670x over previous
import functools

import jax
import jax.numpy as jnp
from jax import lax
from jax.experimental import pallas as pl
from jax.experimental.pallas import tpu as pltpu
from jax.experimental.pallas import tpu_sc as plsc

N = 10000
E = 320000
D_IN = 128
H = 128
OUT = 64
FUS = 128

NC = 2
NS = 16
EPW = E // (NC * NS)
CHUNK = 80
NCHUNK = EPW // CHUNK
NBLK = 5
BCH = NCHUNK // NBLK
NBUF = 5
STEPS = NCHUNK // NBUF
NPAD = 10240
RPT = NPAD // NS

BLK = 1000
GRID = N // BLK


@functools.cache
def _deg_kernel_build():
    mesh = plsc.VectorSubcoreMesh(core_axis_name="c", subcore_axis_name="s")
    return functools.partial(
        pl.kernel,
        out_type=jax.ShapeDtypeStruct((NC * NPAD,), jnp.float32),
        mesh=mesh,
        scratch_types=[
            pltpu.VMEM((NCHUNK, CHUNK), jnp.int32),
            pltpu.VMEM((CHUNK,), jnp.float32),
            pltpu.VMEM_SHARED((NPAD,), jnp.float32),
            pltpu.SemaphoreType.DMA,
        ],
    )(_deg_body)


def _deg_body(dst2_hbm, ones_hbm, zeros_hbm, out_hbm, dst2_v, ones_v, acc_sh,
              ssem):
    c = lax.axis_index("c")
    s = lax.axis_index("s")
    row0 = s * RPT
    pltpu.sync_copy(zeros_hbm, acc_sh.at[pl.ds(row0, RPT)])
    pltpu.sync_copy(ones_hbm, ones_v)
    pltpu.sync_copy(dst2_hbm.at[c * NS + s], dst2_v)
    plsc.subcore_barrier()

    def body(j, carry):
        k0 = j * NCHUNK // STEPS
        descs = [
            pltpu.async_copy(ones_v, acc_sh.at[dst2_v.at[k0 + b]], ssem,
                             add=True)
            for b in range(NCHUNK // STEPS)
        ]
        for d in descs:
            d.wait()
        return carry

    lax.fori_loop(0, STEPS, body, 0)
    plsc.subcore_barrier()
    pltpu.sync_copy(acc_sh.at[pl.ds(row0, RPT)],
                    out_hbm.at[pl.ds(c * NPAD + row0, RPT)])


@functools.cache
def _agg_kernel_build():
    mesh = plsc.VectorSubcoreMesh(core_axis_name="c", subcore_axis_name="s")
    return functools.partial(
        pl.kernel,
        out_type=jax.ShapeDtypeStruct((NC * NPAD, H), jnp.float32),
        mesh=mesh,
        scratch_types=[
            pltpu.VMEM((BCH, CHUNK), jnp.int32),
            pltpu.VMEM((BCH, CHUNK), jnp.int32),
            pltpu.VMEM((CHUNK, H), jnp.float32),
            pltpu.VMEM((CHUNK, H), jnp.float32),
            pltpu.VMEM_SHARED((NPAD, H), jnp.float32),
            pltpu.SemaphoreType.DMA,
            pltpu.SemaphoreType.DMA,
        ],
    )(_agg_body)


def _agg_body(src4_hbm, dst4_hbm, hs_hbm, zeros_hbm, out_hbm,
              src_v, dst_v, r0, r1, acc_sh, g0, g1):
    c = lax.axis_index("c")
    s = lax.axis_index("s")
    row0 = s * RPT
    pltpu.sync_copy(zeros_hbm, acc_sh.at[pl.ds(row0, RPT)])
    wid = c * NS + s
    plsc.subcore_barrier()

    def blk(b, carry):
        pltpu.sync_copy(src4_hbm.at[wid, b], src_v)
        pltpu.sync_copy(dst4_hbm.at[wid, b], dst_v)
        pltpu.async_copy(hs_hbm.at[src_v.at[0]], r0, g0).wait()

        def body(j, c2):
            k = 2 * j
            pltpu.async_copy(hs_hbm.at[src_v.at[k + 1]], r1, g1)
            pltpu.sync_copy(r0, acc_sh.at[dst_v.at[k]], add=True)
            gd0 = pltpu.async_copy(hs_hbm.at[src_v.at[k + 2]], r0, g0)
            pltpu.make_async_copy(hs_hbm.at[src_v.at[k + 1]], r1, g1).wait()
            pltpu.sync_copy(r1, acc_sh.at[dst_v.at[k + 1]], add=True)
            gd0.wait()
            return c2

        lax.fori_loop(0, (BCH - 1) // 2, body, 0)
        pltpu.sync_copy(r0, acc_sh.at[dst_v.at[BCH - 1]], add=True)
        return carry

    lax.fori_loop(0, NBLK, blk, 0)
    plsc.subcore_barrier()
    pltpu.sync_copy(acc_sh.at[pl.ds(row0, RPT)],
                    out_hbm.at[pl.ds(c * NPAD + row0, RPT)])


def _dinv(dega, degb):
    return lax.rsqrt(dega[:, :1] + degb[:, :1] + 1.0)


def _tc1_body(x_ref, dega_ref, degb_ref, w_ref, out_ref):
    dinv = _dinv(dega_ref[...], degb_ref[...])
    out_ref[...] = jnp.dot(x_ref[...], w_ref[...],
                           preferred_element_type=jnp.float32) * dinv


def _tc2_body(a1a_ref, a1b_ref, hs1_ref, dega_ref, degb_ref, gb1_ref, gw2_ref,
              out_ref):
    dinv = _dinv(dega_ref[...], degb_ref[...])
    h1 = jnp.maximum((a1a_ref[...] + a1b_ref[...] + hs1_ref[...]) * dinv
                     + gb1_ref[...], 0.0)
    out_ref[...] = jnp.dot(h1, gw2_ref[...],
                           preferred_element_type=jnp.float32) * dinv


def _tc3_body(a2a_ref, a2b_ref, hs2_ref, dega_ref, degb_ref, fem_ref, topo_ref,
              gb2_ref, fw0a_ref, fw0b_ref, fb0_ref, fw1_ref, fb1_ref,
              fwo_ref, fbo_ref, tw0a_ref, tw0b_ref, tb0_ref, tw1_ref, tb1_ref,
              two_ref, tbo_ref, uwh_ref, uwf_ref, uwt_ref, ub_ref,
              rw1_ref, rb1_ref, rw2_ref, rb2_ref, aw1_ref, ab1_ref,
              aw2_ref, ab2_ref, add_ref, rem_ref):
    mm = functools.partial(jnp.dot, preferred_element_type=jnp.float32)
    dinv = _dinv(dega_ref[...], degb_ref[...])
    h = jnp.maximum((a2a_ref[...] + a2b_ref[...] + hs2_ref[...]) * dinv
                    + gb2_ref[...], 0.0)
    f = jnp.maximum(mm(h, fw0a_ref[...]) + fem_ref[...] * fw0b_ref[...]
                    + fb0_ref[...], 0.0)
    f = jnp.maximum(mm(f, fw1_ref[...]) + fb1_ref[...], 0.0)
    f = mm(f, fwo_ref[...]) + fbo_ref[...]
    t = jnp.maximum(mm(h, tw0a_ref[...]) + topo_ref[...] * tw0b_ref[...]
                    + tb0_ref[...], 0.0)
    t = jnp.maximum(mm(t, tw1_ref[...]) + tb1_ref[...], 0.0)
    t = mm(t, two_ref[...]) + tbo_ref[...]
    fus = jnp.maximum(mm(h, uwh_ref[...]) + mm(f, uwf_ref[...])
                      + mm(t, uwt_ref[...]) + ub_ref[...], 0.0)
    r = jnp.maximum(mm(fus, rw1_ref[...]) + rb1_ref[...], 0.0)
    rem_ref[...] = jnp.tanh(mm(r, rw2_ref[...]) + rb2_ref[...])
    a = jnp.maximum(mm(fus, aw1_ref[...]) + ab1_ref[...], 0.0)
    add_ref[...] = jnp.tanh(mm(a, aw2_ref[...]) + ab2_ref[...])


def _row_spec(width):
    return pl.BlockSpec((BLK, width), lambda i: (i, 0))


def _full_spec(shape):
    return pl.BlockSpec(shape, lambda i: (0,) * len(shape))


def _tc_call(body, ins, row_widths, full_shapes, out_widths):
    in_specs = ([_row_spec(w) for w in row_widths]
                + [_full_spec(s) for s in full_shapes])
    out_specs = [_row_spec(w) for w in out_widths]
    out_shape = [jax.ShapeDtypeStruct((N, w), jnp.float32) for w in out_widths]
    outs = pl.pallas_call(
        body,
        grid=(GRID,),
        in_specs=in_specs,
        out_specs=out_specs if len(out_specs) > 1 else out_specs[0],
        out_shape=out_shape if len(out_shape) > 1 else out_shape[0],
    )(*ins)
    return outs


def kernel(x, edge_index, fem_reward, topology_reward, gW1, gb1, gW2, gb2,
           fW0, fb0, fW1, fb1, fWo, fbo, tW0, tb0, tW1, tb1, tWo, tbo,
           uW, ub, rW1, rb1, rW2, rb2, aW1, ab1, aW2, ab2):
    src2 = edge_index[0].reshape(NC * NS, NCHUNK, CHUNK)
    dst2 = edge_index[1].reshape(NC * NS, NCHUNK, CHUNK)
    src4 = edge_index[0].reshape(NC * NS, NBLK, BCH, CHUNK)
    dst4 = edge_index[1].reshape(NC * NS, NBLK, BCH, CHUNK)
    zeros_h = jnp.zeros((RPT, H), jnp.float32)
    zeros_d = jnp.zeros((RPT,), jnp.float32)
    ones_d = jnp.ones((CHUNK,), jnp.float32)

    deg_parts = _deg_kernel_build()(dst2, ones_d, zeros_d)
    dega = deg_parts[:N].reshape(N, 1)
    degb = deg_parts[NPAD:NPAD + N].reshape(N, 1)

    h1s = _tc_call(_tc1_body, (x, dega, degb, gW1), (D_IN, 1, 1),
                   ((D_IN, H),), (H,))
    agg1 = _agg_kernel_build()(src4, dst4, h1s, zeros_h)
    a1a = agg1[:N]
    a1b = agg1[NPAD:NPAD + N]

    h2s = _tc_call(_tc2_body,
                   (a1a, a1b, h1s, dega, degb, gb1.reshape(1, H), gW2),
                   (H, H, H, 1, 1), ((1, H), (H, H)), (H,))
    agg2 = _agg_kernel_build()(src4, dst4, h2s, zeros_h)
    a2a = agg2[:N]
    a2b = agg2[NPAD:NPAD + N]

    fW0a, fW0b = fW0[:H], fW0[H:H + 1]
    tW0a, tW0b = tW0[:H], tW0[H:H + 1]
    uWh, uWf, uWt = uW[:H], uW[H:H + OUT], uW[H + OUT:]
    rW2p = jnp.pad(rW2, ((0, 0), (0, 7)))
    rb2p = jnp.pad(rb2, (0, 7)).reshape(1, 8)
    aW2p = jnp.pad(aW2, ((0, 0), (0, 5)))
    ab2p = jnp.pad(ab2, (0, 5)).reshape(1, 8)

    add_p, rem_p = _tc_call(
        _tc3_body,
        (a2a, a2b, h2s, dega, degb, fem_reward, topology_reward,
         gb2.reshape(1, H),
         fW0a, fW0b, fb0.reshape(1, H), fW1, fb1.reshape(1, H),
         fWo, fbo.reshape(1, OUT),
         tW0a, tW0b, tb0.reshape(1, H), tW1, tb1.reshape(1, H),
         tWo, tbo.reshape(1, OUT),
         uWh, uWf, uWt, ub.reshape(1, FUS),
         rW1, rb1.reshape(1, FUS), rW2p, rb2p,
         aW1, ab1.reshape(1, FUS), aW2p, ab2p),
        (H, H, H, 1, 1, 1, 1),
        ((1, H), (H, H), (1, H), (1, H), (H, H), (1, H), (H, OUT), (1, OUT),
         (H, H), (1, H), (1, H), (H, H), (1, H), (H, OUT), (1, OUT),
         (H, FUS), (OUT, FUS), (OUT, FUS), (1, FUS),
         (FUS, FUS), (1, FUS), (FUS, 8), (1, 8),
         (FUS, FUS), (1, FUS), (FUS, 8), (1, 8)),
        (8, 8))
    return (add_p[:, :3], rem_p[:, :1])
